# baseline (speedup 1.0000x reference)
import jax
import jax.numpy as jnp
from jax import lax
from jax.experimental import pallas as pl
from jax.experimental.pallas import tpu as pltpu
from jax.experimental.pallas import tpu_sc as plsc

BATCH = 16384
D = 32
NC = 2
NS = 16
L = 16
NW = NC * NS
BPW = BATCH // NW

_mesh = plsc.VectorSubcoreMesh(core_axis_name="c", subcore_axis_name="s")


def _wid():
    return lax.axis_index("s") * NC + lax.axis_index("c")


NSEM = 8
SPW = BPW // NSEM


def _phase1_body(user_hbm, item_hbm, uf_hbm, if_hbm, gu_hbm, gi_hbm,
                 uidx_v, iidx_v, usems, isems):
    wid = _wid()
    base = wid * BPW

    pltpu.sync_copy(user_hbm.at[pl.ds(base, BPW)], uidx_v)
    pltpu.sync_copy(item_hbm.at[pl.ds(base, BPW)], iidx_v)

    def wave(w, carry):
        uvec = uidx_v[pl.ds(w * L, L)]
        ivec = iidx_v[pl.ds(w * L, L)]
        for j in range(L):
            b = base + w * L + j
            q = j % NSEM
            pltpu.async_copy(
                uf_hbm.at[pl.ds(uvec[j], 1)], gu_hbm.at[pl.ds(b, 1)],
                usems.at[q])
            pltpu.async_copy(
                if_hbm.at[pl.ds(ivec[j], 1)], gi_hbm.at[pl.ds(b, 1)],
                isems.at[q])
        return carry

    lax.fori_loop(0, BPW // L, wave, 0)
    for q in range(NSEM):
        pltpu.make_async_copy(
            uf_hbm.at[pl.ds(0, SPW)], gu_hbm.at[pl.ds(base, SPW)],
            usems.at[q]).wait()
        pltpu.make_async_copy(
            if_hbm.at[pl.ds(0, SPW)], gi_hbm.at[pl.ds(base, SPW)],
            isems.at[q]).wait()


def _phase2_body(gu_hbm, gi_hbm, out_hbm, urows_v, irows_v, out_v, sem):
    wid = _wid()
    base = wid * BPW

    ucopy = pltpu.make_async_copy(gu_hbm.at[pl.ds(base, BPW)], urows_v, sem)
    icopy = pltpu.make_async_copy(gi_hbm.at[pl.ds(base, BPW)], irows_v, sem)
    ucopy.start()
    icopy.start()
    ucopy.wait()
    icopy.wait()

    lane = lax.iota(jnp.int32, L)

    def group(g, carry):
        rows = g * L + lane
        acc = jnp.zeros((L,), jnp.float32)
        for k in range(D):
            col = jnp.full((L,), k, jnp.int32)
            uv = plsc.load_gather(urows_v, [rows, col])
            iv = plsc.load_gather(irows_v, [rows, col])
            acc = acc + uv * iv
        out_v[pl.ds(g * L, L)] = acc
        return carry

    lax.fori_loop(0, BPW // L, group, 0)
    pltpu.sync_copy(out_v, out_hbm.at[pl.ds(base, BPW)])


def kernel(user, item, user_factors, item_factors):
    phase1 = pl.kernel(
        _phase1_body,
        out_type=(
            jax.ShapeDtypeStruct((BATCH, D), jnp.float32),
            jax.ShapeDtypeStruct((BATCH, D), jnp.float32),
        ),
        mesh=_mesh,
        compiler_params=pltpu.CompilerParams(
            needs_layout_passes=False, use_tc_tiling_on_sc=True),
        scratch_types=[
            pltpu.VMEM((BPW,), jnp.int32),
            pltpu.VMEM((BPW,), jnp.int32),
            pltpu.SemaphoreType.DMA((NSEM,)),
            pltpu.SemaphoreType.DMA((NSEM,)),
        ],
    )

    phase2 = pl.kernel(
        _phase2_body,
        out_type=jax.ShapeDtypeStruct((BATCH,), jnp.float32),
        mesh=_mesh,
        compiler_params=pltpu.CompilerParams(
            needs_layout_passes=False, use_tc_tiling_on_sc=False),
        scratch_types=[
            pltpu.VMEM((BPW, D), jnp.float32),
            pltpu.VMEM((BPW, D), jnp.float32),
            pltpu.VMEM((BPW,), jnp.float32),
            pltpu.SemaphoreType.DMA,
        ],
    )

    gu, gi = phase1(user.astype(jnp.int32), item.astype(jnp.int32),
                    user_factors, item_factors)
    return phase2(gu, gi)


# speedup vs baseline: 1.2425x; 1.2425x over previous
import jax
import jax.numpy as jnp
from jax import lax
from jax.experimental import pallas as pl
from jax.experimental.pallas import tpu as pltpu
from jax.experimental.pallas import tpu_sc as plsc

BATCH = 16384
D = 32
NC = 2
NS = 16
L = 16
NW = NC * NS
BPW = BATCH // NW
CHUNK = 128
NCH = BPW // CHUNK


def _sc_body(user_hbm, item_hbm, uf_hbm, if_hbm, out_hbm,
             uidx_v, iidx_v, urows_v, irows_v, out_v, sem):
    wid = lax.axis_index("s") * NC + lax.axis_index("c")
    base = wid * BPW

    for j in range(NCH):
        pltpu.sync_copy(user_hbm.at[pl.ds(base + j * CHUNK, CHUNK)], uidx_v.at[j])
        pltpu.sync_copy(item_hbm.at[pl.ds(base + j * CHUNK, CHUNK)], iidx_v.at[j])

    copies = []
    for j in range(NCH):
        copies.append(pltpu.async_copy(
            uf_hbm.at[uidx_v.at[j]], urows_v.at[pl.ds(j * CHUNK, CHUNK)], sem))
        copies.append(pltpu.async_copy(
            if_hbm.at[iidx_v.at[j]], irows_v.at[pl.ds(j * CHUNK, CHUNK)], sem))
    for c in copies:
        c.wait()

    lane = lax.iota(jnp.int32, L)

    def group(g, carry):
        rows = g * L + lane
        acc = jnp.zeros((L,), jnp.float32)
        for k in range(D):
            col = jnp.full((L,), k, jnp.int32)
            uv = plsc.load_gather(urows_v, [rows, col])
            iv = plsc.load_gather(irows_v, [rows, col])
            acc = acc + uv * iv
        out_v[pl.ds(g * L, L)] = acc
        return carry

    lax.fori_loop(0, BPW // L, group, 0)

    pltpu.sync_copy(out_v, out_hbm.at[pl.ds(base, BPW)])


def kernel(user, item, user_factors, item_factors):
    mesh = plsc.VectorSubcoreMesh(core_axis_name="c", subcore_axis_name="s")
    sc_call = pl.kernel(
        _sc_body,
        out_type=jax.ShapeDtypeStruct((BATCH,), jnp.float32),
        mesh=mesh,
        compiler_params=pltpu.CompilerParams(
            needs_layout_passes=False, use_tc_tiling_on_sc=False),
        scratch_types=[
            pltpu.VMEM((NCH, CHUNK), jnp.int32),
            pltpu.VMEM((NCH, CHUNK), jnp.int32),
            pltpu.VMEM((BPW, D), jnp.float32),
            pltpu.VMEM((BPW, D), jnp.float32),
            pltpu.VMEM((BPW,), jnp.float32),
            pltpu.SemaphoreType.DMA,
        ],
    )
    return sc_call(user.astype(jnp.int32), item.astype(jnp.int32),
                   user_factors, item_factors)
